# baseline (speedup 1.0000x reference)
import functools

import jax
import jax.numpy as jnp
from jax import lax
from jax.experimental import pallas as pl
from jax.experimental.pallas import tpu as pltpu
from jax.experimental.pallas import tpu_sc as plsc

VOCAB = 100000
EMBED = 128
MAX_LEN = 200
BATCH = 4096
SEQ = 200

NC = 2
NS = 16
NW = NC * NS
SEQ_PER_W = BATCH // NW
HALF = SEQ // 2

_mesh = plsc.VectorSubcoreMesh(core_axis_name="c", subcore_axis_name="s")


@functools.partial(
    pl.kernel,
    mesh=_mesh,
    out_type=jax.ShapeDtypeStruct((BATCH, SEQ, EMBED), jnp.float32),
    scratch_types=[
        pltpu.VMEM((SEQ_PER_W, 2, HALF), jnp.int32),
        pltpu.VMEM((2, SEQ, EMBED), jnp.float32),
        pltpu.SemaphoreType.DMA,
        pltpu.SemaphoreType.DMA,
        pltpu.SemaphoreType.DMA,
        pltpu.SemaphoreType.DMA,
    ],
)
def _gather_kernel(x_hbm, wt_hbm, out_hbm, idx_v, rows_v,
                   gsem0, gsem1, osem0, osem1):
    wid = lax.axis_index("s") * NC + lax.axis_index("c")
    gsems = (gsem0, gsem1)
    osems = (osem0, osem1)

    pltpu.sync_copy(x_hbm.at[wid], idx_v)

    def issue_gathers(i, b):
        pltpu.async_copy(wt_hbm.at[idx_v.at[i, 0]],
                         rows_v.at[b, pl.ds(0, HALF)], gsems[b])
        pltpu.async_copy(wt_hbm.at[idx_v.at[i, 1]],
                         rows_v.at[b, pl.ds(HALF, HALF)], gsems[b])

    def drain(sem, b):
        pltpu.make_async_copy(wt_hbm.at[pl.ds(0, SEQ)], rows_v.at[b], sem).wait()

    issue_gathers(0, 0)

    def outer_body(k, carry):
        for b in range(2):
            i = 2 * k + b
            drain(gsems[b], b)
            if b == 0:
                @pl.when(k >= 1)
                def _():
                    drain(osems[1], 1)
            else:
                drain(osems[0], 0)
            if b == 0:
                issue_gathers(i + 1, 1)
            else:
                @pl.when(k < (SEQ_PER_W // 2) - 1)
                def _():
                    issue_gathers(i + 1, 0)
            pltpu.async_copy(rows_v.at[b], out_hbm.at[wid * SEQ_PER_W + i],
                             osems[b])
        return carry

    lax.fori_loop(0, SEQ_PER_W // 2, outer_body, 0)
    drain(osems[1], 1)


BBLK = 128


def _add_body(g_ref, p_ref, o_ref):
    o_ref[...] = g_ref[...] + p_ref[...][None]


def _pos_add(gathered, pos_table):
    return pl.pallas_call(
        _add_body,
        grid=(BATCH // BBLK,),
        in_specs=[
            pl.BlockSpec((BBLK, SEQ, EMBED), lambda i: (i, 0, 0)),
            pl.BlockSpec((SEQ, EMBED), lambda i: (0, 0)),
        ],
        out_specs=pl.BlockSpec((BBLK, SEQ, EMBED), lambda i: (i, 0, 0)),
        out_shape=jax.ShapeDtypeStruct((BATCH, SEQ, EMBED), jnp.float32),
    )(gathered, pos_table)


def kernel(x, word_table, pos_table):
    x4 = x.astype(jnp.int32).reshape(NW, SEQ_PER_W, 2, HALF)
    gathered = _gather_kernel(x4, word_table)
    return _pos_add(gathered, pos_table)


# speedup vs baseline: 1.0212x; 1.0212x over previous
import functools

import jax
import jax.numpy as jnp
from jax import lax
from jax.experimental import pallas as pl
from jax.experimental.pallas import tpu as pltpu
from jax.experimental.pallas import tpu_sc as plsc

VOCAB = 100000
EMBED = 128
MAX_LEN = 200
BATCH = 4096
SEQ = 200

NC = 2
NS = 16
NW = NC * NS
GROUP = 2 * SEQ
NGROUPS = BATCH * SEQ // GROUP
HALF = GROUP // 4

NCHUNK = 4
CG = NGROUPS // NCHUNK
GPW = CG // NW

_mesh = plsc.VectorSubcoreMesh(core_axis_name="c", subcore_axis_name="s")


@functools.partial(
    pl.kernel,
    mesh=_mesh,
    out_type=jax.ShapeDtypeStruct((CG, GROUP, EMBED), jnp.float32),
    scratch_types=[
        pltpu.VMEM((GPW, 4, HALF), jnp.int32),
        pltpu.VMEM((2, GROUP, EMBED), jnp.float32),
        pltpu.SemaphoreType.DMA,
        pltpu.SemaphoreType.DMA,
        pltpu.SemaphoreType.DMA,
        pltpu.SemaphoreType.DMA,
    ],
)
def _gather_kernel(x_hbm, wt_hbm, out_hbm, idx_v, rows_v,
                   gsem0, gsem1, osem0, osem1):
    wid = lax.axis_index("s") * NC + lax.axis_index("c")
    gsems = (gsem0, gsem1)
    osems = (osem0, osem1)

    pltpu.sync_copy(x_hbm.at[wid], idx_v)

    def issue_gathers(g, b):
        for j in range(4):
            pltpu.async_copy(wt_hbm.at[idx_v.at[g, j]],
                             rows_v.at[b, pl.ds(j * HALF, HALF)], gsems[b])

    def drain(sem, b):
        pltpu.make_async_copy(wt_hbm.at[pl.ds(0, GROUP)], rows_v.at[b], sem).wait()

    issue_gathers(0, 0)

    def outer_body(k, carry):
        for b in range(2):
            g = 2 * k + b
            drain(gsems[b], b)
            if b == 0:
                @pl.when(k >= 1)
                def _():
                    drain(osems[1], 1)
            else:
                drain(osems[0], 0)
            if b == 0:
                issue_gathers(g + 1, 1)
            else:
                @pl.when(k < (GPW // 2) - 1)
                def _():
                    issue_gathers(g + 1, 0)
            pltpu.async_copy(rows_v.at[b], out_hbm.at[wid * GPW + g],
                             osems[b])
        return carry

    lax.fori_loop(0, GPW // 2, outer_body, 0)
    drain(osems[1], 1)


GBLK = 64
CSTEPS = CG // GBLK


def _add_body_first(g_ref, p_ref, o_ref):
    o_ref[...] = g_ref[...] + p_ref[...][None]


def _add_body_next(g_ref, p_ref, prev_ref, o_ref):
    del prev_ref
    o_ref[...] = g_ref[...] + p_ref[...][None]


def _pos_add_chunk(gathered_c, pos2, prev_out, c):
    out_spec = pl.BlockSpec((GBLK, GROUP, EMBED),
                            lambda i, c=c: (c * CSTEPS + i, 0, 0))
    g_spec = pl.BlockSpec((GBLK, GROUP, EMBED), lambda i: (i, 0, 0))
    p_spec = pl.BlockSpec((GROUP, EMBED), lambda i: (0, 0))
    out_shape = jax.ShapeDtypeStruct((NGROUPS, GROUP, EMBED), jnp.float32)
    if prev_out is None:
        return pl.pallas_call(
            _add_body_first,
            grid=(CSTEPS,),
            in_specs=[g_spec, p_spec],
            out_specs=out_spec,
            out_shape=out_shape,
        )(gathered_c, pos2)
    return pl.pallas_call(
        _add_body_next,
        grid=(CSTEPS,),
        in_specs=[g_spec, p_spec, pl.BlockSpec(memory_space=pl.ANY)],
        out_specs=out_spec,
        out_shape=out_shape,
        input_output_aliases={2: 0},
    )(gathered_c, pos2, prev_out)


def kernel(x, word_table, pos_table):
    x5 = x.astype(jnp.int32).reshape(NCHUNK, NW, GPW, 4, HALF)
    pos2 = jnp.concatenate([pos_table, pos_table], axis=0)
    out = None
    for c in range(NCHUNK):
        gathered_c = _gather_kernel(x5[c], word_table)
        out = _pos_add_chunk(gathered_c, pos2, out, c)
    return out.reshape(BATCH, SEQ, EMBED)
